# baseline (speedup 1.0000x reference)
import functools

import jax
import jax.numpy as jnp
from jax import lax
from jax.experimental import pallas as pl
from jax.experimental.pallas import tpu as pltpu
from jax.experimental.pallas import tpu_sc as plsc

NB = 8
BATCH = 16384
F = 128
L = 16
GPR = F // L
NC, NS = 2, 16
NW = NC * NS
R = 16
SB = 10240
ROWS_TC = BATCH - SB
K24 = 3 * NB
ROWS_W = SB // NW
NCH = ROWS_W // R
NPAIR = NCH // 2
BR = 512

_LN2 = 0.6931471805599453
_SQRT2 = 1.4142135623730951


def _atanh2(z):
    z2 = z * z
    p = 1.0 + z2 * (1.0 / 3.0 + z2 * (1.0 / 5.0 + z2 * (1.0 / 7.0)))
    return 2.0 * z * p


def _atanh2s(z):
    z2 = z * z
    p = 1.0 + z2 * (1.0 / 3.0 + z2 * (1.0 / 5.0))
    return 2.0 * z * p


def _slog(v):
    i = lax.bitcast_convert_type(v, jnp.int32)
    e = lax.shift_right_arithmetic(i, 23) - 127
    m = lax.bitcast_convert_type((i & 0x7FFFFF) | 0x3F800000, jnp.float32)
    big = m > _SQRT2
    m = jnp.where(big, m * 0.5, m)
    ef = (e + jnp.where(big, 1, 0)).astype(jnp.float32)
    z = (m - 1.0) / (m + 1.0)
    return ef * _LN2 + _atanh2s(z)


def _softplus(t):
    a = jnp.exp(-jnp.abs(t))
    return jnp.maximum(t, 0.0) + _atanh2(a / (a + 2.0))


def _lanesum(v):
    lane = lax.iota(jnp.int32, L)
    for sh in (8, 4, 2, 1):
        v = v + v.at[lane ^ sh].get(mode="promise_in_bounds")
    return v


def _spline_group(pv, xv, x0v, dxv, rdxv, softplus=_softplus, logf=_slog):
    wl, hl, sl = pv[0:NB], pv[NB:2 * NB], pv[2 * NB:]
    ew = [jnp.exp(wl[k]) for k in range(NB)]
    eh = [jnp.exp(hl[k]) for k in range(NB)]
    sw = ew[0]
    sh = eh[0]
    for k in range(1, NB):
        sw = sw + ew[k]
        sh = sh + eh[k]
    cw = dxv / sw
    ch = dxv / sh
    one = jnp.ones_like(xv)
    zero = jnp.zeros_like(xv)
    xmx0 = xv - x0v
    xs = xmx0 * sw * rdxv
    cume = ew[0]
    cumh = eh[0]
    cumew_s, ew_s = zero, ew[0]
    cumeh_s, eh_s = zero, eh[0]
    s0l_s, s1l_s = sl[0], sl[0]
    m1 = xs > cume
    m = m1
    for k in range(1, NB):
        if k > 1:
            m = xs > cume
        cumew_s = jnp.where(m, cume, cumew_s)
        ew_s = jnp.where(m, ew[k], ew_s)
        cumeh_s = jnp.where(m, cumh, cumeh_s)
        eh_s = jnp.where(m, eh[k], eh_s)
        s0l_s = jnp.where(m, sl[k - 1], s0l_s)
        if k < NB - 1:
            s1l_s = jnp.where(m, sl[k], s1l_s)
            cume = cume + ew[k]
            cumh = cumh + eh[k]
    m7 = m
    s0_s = jnp.where(m1, softplus(s0l_s), one)
    s1_s = jnp.where(m7, one, softplus(s1l_s))
    w_s = ew_s * cw
    h_s = eh_s * ch
    ky_s = x0v + cumeh_s * ch
    rw = 1.0 / w_s
    s = h_s * rw
    eps = (xmx0 - cumew_s * cw) * rw
    omeps = 1.0 - eps
    e1 = eps * omeps
    e2 = eps * eps
    s2 = s + s
    numer = h_s * (s * e2 + s0_s * e1)
    denom = s + (s1_s + s0_s - s2) * e1
    rden = 1.0 / denom
    y = ky_s + numer * rden
    num2 = (s * s) * (s1_s * e2 + s2 * e1 + s0_s * (omeps * omeps))
    dy = num2 * (rden * rden)
    return y, logf(dy)


def _sc_body(x_hbm, p_hbm, x0_hbm, xf_hbm, y_hbm, ld_hbm,
             xb, pb, yb, x0b, dxb, rdxb, ldb,
             psem0, psem1, xsem0, xsem1, ysem0, ysem1):
    wid = lax.axis_index("s") * NC + lax.axis_index("c")
    base = wid * ROWS_W
    pltpu.sync_copy(x0_hbm, x0b)
    pltpu.sync_copy(xf_hbm, dxb)
    for g in range(GPR):
        sl16 = pl.ds(g * L, L)
        dxv = dxb[sl16] - x0b[sl16]
        dxb[sl16] = dxv
        rdxb[sl16] = 1.0 / dxv
    lane = lax.iota(jnp.int32, L)

    def compute_chunk(c, pbuf, xbuf, ybuf):
        def row_body(r, ldacc):
            def grp_body(g, acc):
                sl16 = pl.ds(g * L, L)
                xv = xbuf[r, sl16]
                pv = [pbuf[r, k, sl16] for k in range(3 * NB - 1)]
                y, lt = _spline_group(pv, xv, x0b[sl16], dxb[sl16], rdxb[sl16])
                ybuf[r, sl16] = y
                return acc + lt

            acc = lax.fori_loop(0, GPR, grp_body,
                                jnp.zeros((L,), jnp.float32), unroll=8)
            tot = _lanesum(acc)
            return jnp.where(lane == r, tot, ldacc)

        ldacc = lax.fori_loop(0, R, row_body, jnp.zeros((L,), jnp.float32))
        ldb[pl.ds(c * R, R)] = ldacc

    def start_in(row0, pdst, xdst, psem, xsem):
        pltpu.make_async_copy(p_hbm.at[pl.ds(row0, R)], pdst, psem).start()
        pltpu.make_async_copy(x_hbm.at[pl.ds(row0, R)], xdst, xsem).start()

    def wait_in(row0, pdst, xdst, psem, xsem):
        pltpu.make_async_copy(p_hbm.at[pl.ds(row0, R)], pdst, psem).wait()
        pltpu.make_async_copy(x_hbm.at[pl.ds(row0, R)], xdst, xsem).wait()

    start_in(base, pb.at[0], xb.at[0], psem0, xsem0)

    def pair_body(t, carry):
        c0 = t * 2
        c1 = c0 + 1
        row0 = base + c0 * R
        row1 = base + c1 * R
        start_in(row1, pb.at[1], xb.at[1], psem1, xsem1)
        wait_in(row0, pb.at[0], xb.at[0], psem0, xsem0)

        @pl.when(t > 0)
        def _():
            pltpu.make_async_copy(
                yb.at[0], y_hbm.at[pl.ds(row0 - 2 * R, R)], ysem0).wait()

        compute_chunk(c0, pb.at[0], xb.at[0], yb.at[0])
        pltpu.make_async_copy(yb.at[0], y_hbm.at[pl.ds(row0, R)], ysem0).start()

        @pl.when(t + 1 < NPAIR)
        def _():
            start_in(row1 + R, pb.at[0], xb.at[0], psem0, xsem0)

        wait_in(row1, pb.at[1], xb.at[1], psem1, xsem1)

        @pl.when(t > 0)
        def _():
            pltpu.make_async_copy(
                yb.at[1], y_hbm.at[pl.ds(row1 - 2 * R, R)], ysem1).wait()

        compute_chunk(c1, pb.at[1], xb.at[1], yb.at[1])
        pltpu.make_async_copy(yb.at[1], y_hbm.at[pl.ds(row1, R)], ysem1).start()
        return carry

    lax.fori_loop(0, NPAIR, pair_body, 0)
    pltpu.make_async_copy(
        yb.at[0], y_hbm.at[pl.ds(base + (NCH - 2) * R, R)], ysem0).wait()
    pltpu.make_async_copy(
        yb.at[1], y_hbm.at[pl.ds(base + (NCH - 1) * R, R)], ysem1).wait()
    pltpu.sync_copy(ldb, ld_hbm.at[pl.ds(base, ROWS_W)])


@functools.lru_cache(maxsize=1)
def _build_sc_kernel():
    return functools.partial(
        pl.kernel,
        out_type=(jax.ShapeDtypeStruct((SB, F), jnp.float32),
                  jax.ShapeDtypeStruct((SB,), jnp.float32)),
        mesh=plsc.VectorSubcoreMesh(core_axis_name="c", subcore_axis_name="s"),
        compiler_params=pltpu.CompilerParams(use_tc_tiling_on_sc=True),
        scratch_types=[
            pltpu.VMEM((2, R, F), jnp.float32),
            pltpu.VMEM((2, R, K24, F), jnp.float32),
            pltpu.VMEM((2, R, F), jnp.float32),
            pltpu.VMEM((F,), jnp.float32),
            pltpu.VMEM((F,), jnp.float32),
            pltpu.VMEM((F,), jnp.float32),
            pltpu.VMEM((ROWS_W,), jnp.float32),
            pltpu.SemaphoreType.DMA,
            pltpu.SemaphoreType.DMA,
            pltpu.SemaphoreType.DMA,
            pltpu.SemaphoreType.DMA,
            pltpu.SemaphoreType.DMA,
            pltpu.SemaphoreType.DMA,
        ],
    )(_sc_body)


def _softplus_tc(t):
    a = jnp.exp(-jnp.abs(t))
    return jnp.maximum(t, 0.0) + jnp.log(1.0 + a)


def _tc_body(x_ref, p_ref, x0_ref, xf_ref, y_ref, ld_ref):
    xv = x_ref[...]
    x0v = x0_ref[...]
    dxv = xf_ref[...] - x0v
    rdxv = 1.0 / dxv
    pv = [p_ref[:, k, :] for k in range(3 * NB - 1)]
    y, lt = _spline_group(pv, xv, x0v, dxv, rdxv,
                          softplus=_softplus_tc, logf=jnp.log)
    y_ref[...] = y
    ld_ref[...] = jnp.sum(lt, axis=1).reshape(1, 1, BR)


@functools.lru_cache(maxsize=1)
def _build_tc_kernel():
    nblk = ROWS_TC // BR
    return pl.pallas_call(
        _tc_body,
        grid=(nblk,),
        in_specs=[
            pl.BlockSpec((BR, F), lambda i: (i, 0)),
            pl.BlockSpec((BR, K24, F), lambda i: (i, 0, 0)),
            pl.BlockSpec((1, F), lambda i: (0, 0)),
            pl.BlockSpec((1, F), lambda i: (0, 0)),
        ],
        out_specs=[pl.BlockSpec((BR, F), lambda i: (i, 0)),
                   pl.BlockSpec((1, 1, BR), lambda i: (i, 0, 0))],
        out_shape=[jax.ShapeDtypeStruct((ROWS_TC, F), jnp.float32),
                   jax.ShapeDtypeStruct((nblk, 1, BR), jnp.float32)],
    )


def kernel(x, parameters, x0, xf):
    zsc = jnp.zeros((SB, 1, F), jnp.float32)
    p_sc = jnp.concatenate([parameters[ROWS_TC:], zsc], axis=1)
    y_sc, ld_sc = _build_sc_kernel()(x[ROWS_TC:], p_sc, x0, xf)
    ztc = jnp.zeros((ROWS_TC, 1, F), jnp.float32)
    p_tc = jnp.concatenate([parameters[:ROWS_TC], ztc], axis=1)
    y_tc, ld3 = _build_tc_kernel()(
        x, p_tc, x0.reshape(1, F), xf.reshape(1, F))
    y = jnp.concatenate([y_tc, y_sc], axis=0)
    ld = jnp.concatenate([ld3.reshape(ROWS_TC), ld_sc], axis=0)
    return y, ld


# speedup vs baseline: 1.3670x; 1.3670x over previous
import functools

import jax
import jax.numpy as jnp
from jax import lax
from jax.experimental import pallas as pl
from jax.experimental.pallas import tpu as pltpu
from jax.experimental.pallas import tpu_sc as plsc

NB = 8
BATCH = 16384
F = 128
L = 16
GPR = F // L
NC, NS = 2, 16
NW = NC * NS
R = 16
SB = 9216
ROWS_TC = BATCH - SB
ROWS_W = SB // NW
NCH = ROWS_W // R
NPAIR = NCH // 2
BR = 512

_LN2 = 0.6931471805599453
_SQRT2 = 1.4142135623730951


def _atanh2(z):
    z2 = z * z
    p = 1.0 + z2 * (1.0 / 3.0 + z2 * (1.0 / 5.0 + z2 * (1.0 / 7.0)))
    return 2.0 * z * p


def _atanh2s(z):
    z2 = z * z
    p = 1.0 + z2 * (1.0 / 3.0 + z2 * (1.0 / 5.0))
    return 2.0 * z * p


def _slog(v):
    i = lax.bitcast_convert_type(v, jnp.int32)
    e = lax.shift_right_arithmetic(i, 23) - 127
    m = lax.bitcast_convert_type((i & 0x7FFFFF) | 0x3F800000, jnp.float32)
    big = m > _SQRT2
    m = jnp.where(big, m * 0.5, m)
    ef = (e + jnp.where(big, 1, 0)).astype(jnp.float32)
    z = (m - 1.0) / (m + 1.0)
    return ef * _LN2 + _atanh2s(z)


def _softplus(t):
    a = jnp.exp(-jnp.abs(t))
    return jnp.maximum(t, 0.0) + _atanh2(a / (a + 2.0))


def _lanesum(v):
    lane = lax.iota(jnp.int32, L)
    for sh in (8, 4, 2, 1):
        v = v + v.at[lane ^ sh].get(mode="promise_in_bounds")
    return v


def _spline_group(pv, xv, x0v, dxv, rdxv, softplus=_softplus, logf=_slog):
    wl, hl, sl = pv[0:NB], pv[NB:2 * NB], pv[2 * NB:]
    ew = [jnp.exp(wl[k]) for k in range(NB)]
    eh = [jnp.exp(hl[k]) for k in range(NB)]
    sw = ew[0]
    sh = eh[0]
    for k in range(1, NB):
        sw = sw + ew[k]
        sh = sh + eh[k]
    cw = dxv / sw
    ch = dxv / sh
    one = jnp.ones_like(xv)
    zero = jnp.zeros_like(xv)
    xmx0 = xv - x0v
    xs = xmx0 * sw * rdxv
    cume = ew[0]
    cumh = eh[0]
    cumew_s, ew_s = zero, ew[0]
    cumeh_s, eh_s = zero, eh[0]
    s0l_s, s1l_s = sl[0], sl[0]
    m1 = xs > cume
    m = m1
    for k in range(1, NB):
        if k > 1:
            m = xs > cume
        cumew_s = jnp.where(m, cume, cumew_s)
        ew_s = jnp.where(m, ew[k], ew_s)
        cumeh_s = jnp.where(m, cumh, cumeh_s)
        eh_s = jnp.where(m, eh[k], eh_s)
        s0l_s = jnp.where(m, sl[k - 1], s0l_s)
        if k < NB - 1:
            s1l_s = jnp.where(m, sl[k], s1l_s)
            cume = cume + ew[k]
            cumh = cumh + eh[k]
    m7 = m
    s0_s = jnp.where(m1, softplus(s0l_s), one)
    s1_s = jnp.where(m7, one, softplus(s1l_s))
    w_s = ew_s * cw
    h_s = eh_s * ch
    ky_s = x0v + cumeh_s * ch
    rw = 1.0 / w_s
    s = h_s * rw
    eps = (xmx0 - cumew_s * cw) * rw
    omeps = 1.0 - eps
    e1 = eps * omeps
    e2 = eps * eps
    s2 = s + s
    numer = h_s * (s * e2 + s0_s * e1)
    denom = s + (s1_s + s0_s - s2) * e1
    rden = 1.0 / denom
    y = ky_s + numer * rden
    num2 = (s * s) * (s1_s * e2 + s2 * e1 + s0_s * (omeps * omeps))
    dy = num2 * (rden * rden)
    return y, logf(dy)


def _sc_body(x_hbm, p_hbm, x0_hbm, xf_hbm, y_hbm, ld_hbm,
             xb, pb, yb, x0b, dxb, rdxb, ldb,
             psem0, psem1, xsem0, xsem1, ysem0, ysem1):
    wid = lax.axis_index("s") * NC + lax.axis_index("c")
    base = wid * ROWS_W
    pltpu.sync_copy(x0_hbm, x0b)
    pltpu.sync_copy(xf_hbm, dxb)
    for g in range(GPR):
        sl16 = pl.ds(g * L, L)
        dxv = dxb[sl16] - x0b[sl16]
        dxb[sl16] = dxv
        rdxb[sl16] = 1.0 / dxv
    lane = lax.iota(jnp.int32, L)

    def compute_chunk(c, pbuf, xbuf, ybuf):
        def row_body(r, ldacc):
            def grp_body(g, acc):
                sl16 = pl.ds(g * L, L)
                xv = xbuf[r, sl16]
                pv = [pbuf[r, k, sl16] for k in range(3 * NB - 1)]
                y, lt = _spline_group(pv, xv, x0b[sl16], dxb[sl16], rdxb[sl16])
                ybuf[r, sl16] = y
                return acc + lt

            acc = lax.fori_loop(0, GPR, grp_body,
                                jnp.zeros((L,), jnp.float32), unroll=8)
            tot = _lanesum(acc)
            return jnp.where(lane == r, tot, ldacc)

        ldacc = lax.fori_loop(0, R, row_body, jnp.zeros((L,), jnp.float32))
        ldb[pl.ds(c * R, R)] = ldacc

    def start_in(row0, pdst, xdst, psem, xsem):
        src = ROWS_TC + row0
        pltpu.make_async_copy(p_hbm.at[pl.ds(src, R)], pdst, psem).start()
        pltpu.make_async_copy(x_hbm.at[pl.ds(src, R)], xdst, xsem).start()

    def wait_in(row0, pdst, xdst, psem, xsem):
        src = ROWS_TC + row0
        pltpu.make_async_copy(p_hbm.at[pl.ds(src, R)], pdst, psem).wait()
        pltpu.make_async_copy(x_hbm.at[pl.ds(src, R)], xdst, xsem).wait()

    start_in(base, pb.at[0], xb.at[0], psem0, xsem0)

    def pair_body(t, carry):
        c0 = t * 2
        c1 = c0 + 1
        row0 = base + c0 * R
        row1 = base + c1 * R
        start_in(row1, pb.at[1], xb.at[1], psem1, xsem1)
        wait_in(row0, pb.at[0], xb.at[0], psem0, xsem0)

        @pl.when(t > 0)
        def _():
            pltpu.make_async_copy(
                yb.at[0], y_hbm.at[pl.ds(row0 - 2 * R, R)], ysem0).wait()

        compute_chunk(c0, pb.at[0], xb.at[0], yb.at[0])
        pltpu.make_async_copy(yb.at[0], y_hbm.at[pl.ds(row0, R)], ysem0).start()

        @pl.when(t + 1 < NPAIR)
        def _():
            start_in(row1 + R, pb.at[0], xb.at[0], psem0, xsem0)

        wait_in(row1, pb.at[1], xb.at[1], psem1, xsem1)

        @pl.when(t > 0)
        def _():
            pltpu.make_async_copy(
                yb.at[1], y_hbm.at[pl.ds(row1 - 2 * R, R)], ysem1).wait()

        compute_chunk(c1, pb.at[1], xb.at[1], yb.at[1])
        pltpu.make_async_copy(yb.at[1], y_hbm.at[pl.ds(row1, R)], ysem1).start()
        return carry

    lax.fori_loop(0, NPAIR, pair_body, 0)
    pltpu.make_async_copy(
        yb.at[0], y_hbm.at[pl.ds(base + (NCH - 2) * R, R)], ysem0).wait()
    pltpu.make_async_copy(
        yb.at[1], y_hbm.at[pl.ds(base + (NCH - 1) * R, R)], ysem1).wait()
    pltpu.sync_copy(ldb, ld_hbm.at[pl.ds(base, ROWS_W)])


@functools.lru_cache(maxsize=1)
def _build_sc_kernel():
    return functools.partial(
        pl.kernel,
        out_type=(jax.ShapeDtypeStruct((SB, F), jnp.float32),
                  jax.ShapeDtypeStruct((SB,), jnp.float32)),
        mesh=plsc.VectorSubcoreMesh(core_axis_name="c", subcore_axis_name="s"),
        compiler_params=pltpu.CompilerParams(use_tc_tiling_on_sc=True),
        scratch_types=[
            pltpu.VMEM((2, R, F), jnp.float32),
            pltpu.VMEM((2, R, 3 * NB - 1, F), jnp.float32),
            pltpu.VMEM((2, R, F), jnp.float32),
            pltpu.VMEM((F,), jnp.float32),
            pltpu.VMEM((F,), jnp.float32),
            pltpu.VMEM((F,), jnp.float32),
            pltpu.VMEM((ROWS_W,), jnp.float32),
            pltpu.SemaphoreType.DMA,
            pltpu.SemaphoreType.DMA,
            pltpu.SemaphoreType.DMA,
            pltpu.SemaphoreType.DMA,
            pltpu.SemaphoreType.DMA,
            pltpu.SemaphoreType.DMA,
        ],
    )(_sc_body)


def _softplus_tc(t):
    a = jnp.exp(-jnp.abs(t))
    return jnp.maximum(t, 0.0) + jnp.log(1.0 + a)


def _tc_body(x_ref, p_ref, x0_ref, xf_ref, y_ref, ld_ref):
    xv = x_ref[...]
    x0v = x0_ref[...]
    dxv = xf_ref[...] - x0v
    rdxv = 1.0 / dxv
    pv = [p_ref[:, k, :] for k in range(3 * NB - 1)]
    y, lt = _spline_group(pv, xv, x0v, dxv, rdxv,
                          softplus=_softplus_tc, logf=jnp.log)
    y_ref[...] = y
    ld_ref[...] = jnp.sum(lt, axis=1).reshape(1, 1, BR)


@functools.lru_cache(maxsize=1)
def _build_tc_kernel():
    nblk = ROWS_TC // BR
    return pl.pallas_call(
        _tc_body,
        grid=(nblk,),
        in_specs=[
            pl.BlockSpec((BR, F), lambda i: (i, 0)),
            pl.BlockSpec((BR, 3 * NB - 1, F), lambda i: (i, 0, 0)),
            pl.BlockSpec((1, F), lambda i: (0, 0)),
            pl.BlockSpec((1, F), lambda i: (0, 0)),
        ],
        out_specs=[pl.BlockSpec((BR, F), lambda i: (i, 0)),
                   pl.BlockSpec((1, 1, BR), lambda i: (i, 0, 0))],
        out_shape=[jax.ShapeDtypeStruct((ROWS_TC, F), jnp.float32),
                   jax.ShapeDtypeStruct((nblk, 1, BR), jnp.float32)],
    )


def kernel(x, parameters, x0, xf):
    y_sc, ld_sc = _build_sc_kernel()(x, parameters, x0, xf)
    y_tc, ld3 = _build_tc_kernel()(
        x, parameters, x0.reshape(1, F), xf.reshape(1, F))
    y = jnp.concatenate([y_tc, y_sc], axis=0)
    ld = jnp.concatenate([ld3.reshape(ROWS_TC), ld_sc], axis=0)
    return y, ld
